# baseline (speedup 1.0000x reference)
import functools

import jax
import jax.numpy as jnp
from jax import lax
from jax.experimental import pallas as pl
from jax.experimental.pallas import tpu as pltpu
from jax.experimental.pallas import tpu_sc as plsc

VOCAB = 100000
DIM = 128
NTOK = 4096
TW = 50
B = NTOK * TW

_info = plsc.get_sparse_core_info()
NC = _info.num_cores
NS = _info.num_subcores
NW = NC * NS
RPW = NTOK // NW
BPW = B // NW
CR = 8
CH = CR * TW
NCH = RPW // CR
NBUF = 2
LAG = 1

_mesh = plsc.VectorSubcoreMesh(core_axis_name="c", subcore_axis_name="s")


@functools.partial(
    pl.kernel,
    mesh=_mesh,
    out_type=jax.ShapeDtypeStruct((B, DIM), jnp.float32),
    scratch_types=[
        pltpu.VMEM((RPW, TW), jnp.int32),
        pltpu.VMEM((NBUF, CH, DIM), jnp.float32),
    ]
    + [pltpu.SemaphoreType.DMA] * (2 * NBUF),
)
def _embed_lookup(tok_hbm, table_hbm, out_hbm, idx_v, rows_v, *sems):
    gsems = sems[:NBUF]
    ssems = sems[NBUF:]
    wid = lax.axis_index("s") * NC + lax.axis_index("c")
    base = wid * BPW
    pltpu.sync_copy(tok_hbm.at[pl.ds(wid * RPW, RPW)], idx_v)

    def gather(j, b):
        for q in range(CR):
            pltpu.async_copy(
                table_hbm.at[idx_v.at[j * CR + q]],
                rows_v.at[b, pl.ds(q * TW, TW)],
                gsems[b],
            )

    def wait_gather(b):
        pltpu.make_async_copy(
            table_hbm.at[pl.ds(0, CH)], rows_v.at[b], gsems[b]
        ).wait()

    def store(j, b):
        pltpu.async_copy(
            rows_v.at[b], out_hbm.at[pl.ds(base + j * CH, CH)], ssems[b]
        )

    def wait_store(b):
        pltpu.make_async_copy(
            rows_v.at[b], out_hbm.at[pl.ds(base, CH)], ssems[b]
        ).wait()

    for b in range(LAG):
        gather(b, b)

    for t in range(NBUF):
        wait_gather(t)
        store(t, t)
        b2 = (t + LAG) % NBUF
        if t >= NBUF - LAG:
            wait_store(b2)
        gather(t + LAG, b2)

    def outer(t0, carry):
        tb = t0 * NBUF
        for db in range(NBUF):
            t = tb + db
            wait_gather(db)
            store(t, db)
            b2 = (db + LAG) % NBUF
            wait_store(b2)
            gather(t + LAG, b2)
        return carry

    lax.fori_loop(1, NCH // NBUF - 1, outer, 0)

    for t in range(NCH - NBUF, NCH):
        db = t % NBUF
        wait_gather(db)
        store(t, db)
        if t + LAG < NCH:
            b2 = (db + LAG) % NBUF
            wait_store(b2)
            gather(t + LAG, b2)

    for b in range(NBUF):
        wait_store(b)


def kernel(tokens, embed_table):
    return _embed_lookup(tokens, embed_table)


# speedup vs baseline: 1.0035x; 1.0035x over previous
import functools

import jax
import jax.numpy as jnp
from jax import lax
from jax.experimental import pallas as pl
from jax.experimental.pallas import tpu as pltpu
from jax.experimental.pallas import tpu_sc as plsc

VOCAB = 100000
DIM = 128
B = 4096 * 50

_info = plsc.get_sparse_core_info()
NC = _info.num_cores
NS = _info.num_subcores
NW = NC * NS
BPW = B // NW
IW = 128
GPC = 2
CH = IW * GPC
NCH = BPW // CH
NBUF = 3
LAG = 2

_mesh = plsc.VectorSubcoreMesh(core_axis_name="c", subcore_axis_name="s")


@functools.partial(
    pl.kernel,
    mesh=_mesh,
    out_type=jax.ShapeDtypeStruct((B, DIM), jnp.float32),
    scratch_types=[
        pltpu.VMEM((NCH * GPC, IW), jnp.int32),
        pltpu.VMEM((NBUF, CH, DIM), jnp.float32),
    ]
    + [pltpu.SemaphoreType.DMA] * (2 * NBUF),
)
def _embed_lookup(idx_hbm, table_hbm, out_hbm, idx_v, rows_v, *sems):
    gsems = sems[:NBUF]
    ssems = sems[NBUF:]
    wid = lax.axis_index("s") * NC + lax.axis_index("c")
    base = wid * BPW
    pltpu.sync_copy(idx_hbm.at[wid], idx_v)

    def gather(j, b):
        for q in range(GPC):
            pltpu.async_copy(
                table_hbm.at[idx_v.at[j * GPC + q]],
                rows_v.at[b, pl.ds(q * IW, IW)],
                gsems[b],
            )

    def wait_gather(b):
        pltpu.make_async_copy(
            table_hbm.at[pl.ds(0, CH)], rows_v.at[b], gsems[b]
        ).wait()

    def store(j, b):
        pltpu.async_copy(
            rows_v.at[b], out_hbm.at[pl.ds(base + j * CH, CH)], ssems[b]
        )

    def wait_store(b):
        pltpu.make_async_copy(
            rows_v.at[b], out_hbm.at[pl.ds(base, CH)], ssems[b]
        ).wait()

    for b in range(LAG):
        gather(b, b)

    for t in range(NBUF):
        wait_gather(t)
        store(t, t)
        b2 = (t + LAG) % NBUF
        if t >= NBUF - LAG:
            wait_store(b2)
        gather(t + LAG, b2)

    def outer(t0, carry):
        tb = t0 * NBUF
        for db in range(NBUF):
            t = tb + db
            wait_gather(db)
            store(t, db)
            b2 = (db + LAG) % NBUF
            wait_store(b2)
            gather(t + LAG, b2)
        return carry

    NSTEADY = (NCH - LAG - NBUF) // NBUF
    lax.fori_loop(1, 1 + NSTEADY, outer, 0)

    for t in range((1 + NSTEADY) * NBUF, NCH):
        db = t % NBUF
        wait_gather(db)
        store(t, db)
        if t + LAG < NCH:
            b2 = (db + LAG) % NBUF
            wait_store(b2)
            gather(t + LAG, b2)

    for b in range(NBUF):
        wait_store(b)


def kernel(tokens, embed_table):
    idx = tokens.reshape(NW, NCH * GPC, IW).astype(jnp.int32)
    return _embed_lookup(idx, embed_table)


# speedup vs baseline: 1.0287x; 1.0251x over previous
import functools

import jax
import jax.numpy as jnp
from jax import lax
from jax.experimental import pallas as pl
from jax.experimental.pallas import tpu as pltpu
from jax.experimental.pallas import tpu_sc as plsc

VOCAB = 100000
DIM = 128
NTOK = 4096
TW = 50
B = NTOK * TW

_info = plsc.get_sparse_core_info()
NC = _info.num_cores
NS = _info.num_subcores
NW = NC * NS
RPW = NTOK // NW
BPW = B // NW
CR = 4
CH = CR * TW
NCH = RPW // CR
NBUF = 4
LAG = 3

_mesh = plsc.VectorSubcoreMesh(core_axis_name="c", subcore_axis_name="s")


@functools.partial(
    pl.kernel,
    mesh=_mesh,
    out_type=jax.ShapeDtypeStruct((B, DIM), jnp.float32),
    scratch_types=[
        pltpu.VMEM((RPW, TW), jnp.int32),
        pltpu.VMEM((NBUF, CH, DIM), jnp.float32),
    ]
    + [pltpu.SemaphoreType.DMA] * (2 * NBUF),
)
def _embed_lookup(tok_hbm, table_hbm, out_hbm, idx_v, rows_v, *sems):
    gsems = sems[:NBUF]
    ssems = sems[NBUF:]
    wid = lax.axis_index("s") * NC + lax.axis_index("c")
    base = wid * BPW
    pltpu.sync_copy(tok_hbm.at[pl.ds(wid * RPW, RPW)], idx_v)

    def gather(j, b):
        for q in range(CR):
            pltpu.async_copy(
                table_hbm.at[idx_v.at[j * CR + q]],
                rows_v.at[b, pl.ds(q * TW, TW)],
                gsems[b],
            )

    def wait_gather(b):
        pltpu.make_async_copy(
            table_hbm.at[pl.ds(0, CH)], rows_v.at[b], gsems[b]
        ).wait()

    def store(j, b):
        pltpu.async_copy(
            rows_v.at[b], out_hbm.at[pl.ds(base + j * CH, CH)], ssems[b]
        )

    def wait_store(b):
        pltpu.make_async_copy(
            rows_v.at[b], out_hbm.at[pl.ds(base, CH)], ssems[b]
        ).wait()

    for b in range(LAG):
        gather(b, b)

    for t in range(NBUF):
        wait_gather(t)
        store(t, t)
        b2 = (t + LAG) % NBUF
        if t >= NBUF - LAG:
            wait_store(b2)
        gather(t + LAG, b2)

    def outer(t0, carry):
        tb = t0 * NBUF
        for db in range(NBUF):
            t = tb + db
            wait_gather(db)
            store(t, db)
            b2 = (db + LAG) % NBUF
            wait_store(b2)
            gather(t + LAG, b2)
        return carry

    lax.fori_loop(1, NCH // NBUF - 1, outer, 0)

    for t in range(NCH - NBUF, NCH):
        db = t % NBUF
        wait_gather(db)
        store(t, db)
        if t + LAG < NCH:
            b2 = (db + LAG) % NBUF
            wait_store(b2)
            gather(t + LAG, b2)

    for b in range(NBUF):
        wait_store(b)


def kernel(tokens, embed_table):
    return _embed_lookup(tokens, embed_table)
